# baseline (speedup 1.0000x reference)
import functools

import jax
import jax.numpy as jnp
from jax import lax
from jax.experimental import pallas as pl
from jax.experimental.pallas import tpu as pltpu
from jax.experimental.pallas import tpu_sc as plsc

N = 10000
D = 128
E = 320000

NC = 2
NS = 16
NW = NC * NS
CH = 128
CPT = 80
EP = NW * CPT * CH
NPAD = 16
NACC = N + NPAD
RPT = 624
LASTX = N - NS * RPT
ZR = 16
DEGW = 16


def _sc_mesh():
    return plsc.VectorSubcoreMesh(core_axis_name="c", subcore_axis_name="s")


def _zero_vmem(buf, rows, width):
    def row(i, carry):
        def col(j, c2):
            buf[i, pl.ds(j * 16, 16)] = jnp.zeros((16,), jnp.float32)
            return c2
        return lax.fori_loop(0, width // 16, col, carry)
    lax.fori_loop(0, rows, row, 0)


def _zero_acc(zb, acc, rbase, s):
    def zcp(i, carry):
        pltpu.sync_copy(zb, acc.at[pl.ds(rbase + i * ZR, ZR)])
        return carry
    lax.fori_loop(0, RPT // ZR, zcp, 0)

    @pl.when(s == NS - 1)
    def _():
        pltpu.sync_copy(zb, acc.at[pl.ds(rbase + RPT, LASTX)])


def _drain_acc(acc, out_hbm, c, rbase, s):
    pltpu.sync_copy(acc.at[pl.ds(rbase, RPT)],
                    out_hbm.at[c, pl.ds(rbase, RPT)])

    @pl.when(s == NS - 1)
    def _():
        pltpu.sync_copy(acc.at[pl.ds(rbase + RPT, LASTX)],
                        out_hbm.at[c, pl.ds(rbase + RPT, LASTX)])


def _sc_degree(dst2d):
    @functools.partial(
        pl.kernel,
        out_type=jax.ShapeDtypeStruct((NC, N, DEGW), jnp.float32),
        mesh=_sc_mesh(),
        scratch_types=[
            pltpu.VMEM((CPT, CH), jnp.int32),
            pltpu.VMEM((CH, DEGW), jnp.float32),
            pltpu.VMEM((ZR, DEGW), jnp.float32),
            pltpu.VMEM_SHARED((NACC, DEGW), jnp.float32),
            pltpu.SemaphoreType.DMA,
        ],
    )
    def deg_kernel(dst_hbm, out_hbm, didx, ones, zb, acc, sem):
        c = lax.axis_index("c")
        s = lax.axis_index("s")
        wid = c * NS + s
        rbase = s * RPT

        _zero_vmem(zb, ZR, DEGW)

        def ones_row(i, carry):
            ones[i, pl.ds(0, 16)] = jnp.ones((16,), jnp.float32)
            return carry
        lax.fori_loop(0, CH, ones_row, 0)

        pltpu.sync_copy(dst_hbm.at[pl.ds(wid * CPT, CPT)], didx)
        _zero_acc(zb, acc, rbase, s)
        plsc.subcore_barrier()

        def body(p, carry):
            pltpu.sync_copy(ones, acc.at[didx.at[p]], add=True)
            return carry
        lax.fori_loop(0, CPT, body, 0)
        plsc.subcore_barrier()

        _drain_acc(acc, out_hbm, c, rbase, s)

    return deg_kernel(dst2d)


def _sc_scatter(mp, src2d, dst2d):
    @functools.partial(
        pl.kernel,
        out_type=jax.ShapeDtypeStruct((NC, N, D), jnp.float32),
        mesh=_sc_mesh(),
        scratch_types=[
            pltpu.VMEM((CH,), jnp.int32),
            pltpu.VMEM((CH,), jnp.int32),
            pltpu.VMEM((CH,), jnp.int32),
            pltpu.VMEM((CH,), jnp.int32),
            pltpu.VMEM((CH, D), jnp.float32),
            pltpu.VMEM((CH, D), jnp.float32),
            pltpu.VMEM((ZR, D), jnp.float32),
            pltpu.VMEM_SHARED((NACC, D), jnp.float32),
            pltpu.SemaphoreType.DMA,
            pltpu.SemaphoreType.DMA,
            pltpu.SemaphoreType.DMA,
            pltpu.SemaphoreType.DMA,
        ],
    )
    def scat_kernel(mp_hbm, src_hbm, dst_hbm, out_hbm,
                    si0, si1, di0, di1, r0, r1, zb, acc, gs0, gs1, ss0, ss1):
        c = lax.axis_index("c")
        s = lax.axis_index("s")
        wid = c * NS + s
        rbase = s * RPT

        _zero_vmem(zb, ZR, D)
        _zero_acc(zb, acc, rbase, s)
        plsc.subcore_barrier()

        def load_idx(j, sib, dib):
            pltpu.sync_copy(src_hbm.at[wid * CPT + j], sib)
            pltpu.sync_copy(dst_hbm.at[wid * CPT + j], dib)

        def gather(rb, sib, gs):
            pltpu.async_copy(mp_hbm.at[sib], rb, gs)

        def handle(j, rb, sib, dib, gs, ss, start_next):
            del ss
            pltpu.make_async_copy(mp_hbm.at[sib], rb, gs).wait()
            pltpu.sync_copy(rb, acc.at[dib], add=True)
            if start_next:
                load_idx(j + 2, sib, dib)
                gather(rb, sib, gs)

        load_idx(0, si0, di0)
        gather(r0, si0, gs0)
        load_idx(1, si1, di1)
        gather(r1, si1, gs1)

        def body(p, carry):
            j = 2 * p
            handle(j, r0, si0, di0, gs0, ss0, True)
            handle(j + 1, r1, si1, di1, gs1, ss1, True)
            return carry
        lax.fori_loop(0, CPT // 2 - 1, body, 0)
        handle(CPT - 2, r0, si0, di0, gs0, ss0, False)
        handle(CPT - 1, r1, si1, di1, gs1, ss1, False)
        plsc.subcore_barrier()

        _drain_acc(acc, out_hbm, c, rbase, s)

    return scat_kernel(mp, src2d, dst2d)


def _dinv_from(degp_ref):
    deg = degp_ref[0, :, 0] + degp_ref[1, :, 0] + 1.0
    return lax.rsqrt(deg)


def _tc_first(x, gamma, beta, W1, degp):
    def body(x_ref, g_ref, b_ref, w_ref, degp_ref, out_ref):
        xv = x_ref[...]
        mean = jnp.mean(xv, axis=0, keepdims=True)
        var = jnp.mean((xv - mean) ** 2, axis=0, keepdims=True)
        h = (xv - mean) / jnp.sqrt(var + 1e-5) * g_ref[...] + b_ref[...]
        dinv = _dinv_from(degp_ref)
        xw = jnp.dot(h, w_ref[...], preferred_element_type=jnp.float32)
        out_ref[...] = xw * dinv[:, None]

    return pl.pallas_call(
        body,
        out_shape=jax.ShapeDtypeStruct((N, D), jnp.float32),
    )(x, gamma.reshape(1, D), beta.reshape(1, D), W1, degp)


def _tc_mid(acc, mp, degp, b, W2):
    def body(acc_ref, mp_ref, degp_ref, b_ref, w_ref, out_ref):
        dinv = _dinv_from(degp_ref)
        tot = acc_ref[0] + acc_ref[1] + mp_ref[...]
        h = jnp.maximum(tot * dinv[:, None] + b_ref[...], 0.0)
        xw = jnp.dot(h, w_ref[...], preferred_element_type=jnp.float32)
        out_ref[...] = xw * dinv[:, None]

    return pl.pallas_call(
        body,
        out_shape=jax.ShapeDtypeStruct((N, D), jnp.float32),
    )(acc, mp, degp, b.reshape(1, D), W2)


def _tc_final(acc, mp, degp, b2, Wc1, bc1, Wc2, bc2):
    def body(acc_ref, mp_ref, degp_ref, b2_ref, wc1_ref, bc1_ref,
             wc2_ref, bc2_ref, out_ref):
        dinv = _dinv_from(degp_ref)
        tot = acc_ref[0] + acc_ref[1] + mp_ref[...]
        h = jnp.maximum(tot * dinv[:, None] + b2_ref[...], 0.0)
        hc = jnp.dot(h, wc1_ref[...], preferred_element_type=jnp.float32)
        hc = jnp.maximum(hc + bc1_ref[...], 0.0)
        out = jnp.dot(hc, wc2_ref[...], preferred_element_type=jnp.float32)
        out_ref[...] = out + bc2_ref[...]

    return pl.pallas_call(
        body,
        out_shape=jax.ShapeDtypeStruct((N, 2), jnp.float32),
    )(acc, mp, degp, b2.reshape(1, D), Wc1, bc1.reshape(1, 16),
      Wc2, bc2.reshape(1, 2))


def kernel(x, edge_index, bn_gamma, bn_beta, W1, b1, W2, b2, Wc1, bc1, Wc2, bc2):
    src = edge_index[0].astype(jnp.int32)
    dst = edge_index[1].astype(jnp.int32)

    pad = EP - E
    psrc = jnp.zeros((pad,), jnp.int32)
    pdst = N + (jnp.arange(pad, dtype=jnp.int32) % NPAD)
    src2d = jnp.concatenate([src, psrc]).reshape(EP // CH, CH)
    dst2d = jnp.concatenate([dst, pdst]).reshape(EP // CH, CH)

    degp = _sc_degree(dst2d)
    mp1 = _tc_first(x, bn_gamma, bn_beta, W1, degp)
    acc1 = _sc_scatter(mp1, src2d, dst2d)
    mp2 = _tc_mid(acc1, mp1, degp, b1, W2)
    acc2 = _sc_scatter(mp2, src2d, dst2d)
    return _tc_final(acc2, mp2, degp, b2, Wc1, bc1, Wc2, bc2)


# speedup vs baseline: 3.0179x; 3.0179x over previous
import functools

import jax
import jax.numpy as jnp
from jax import lax
from jax.experimental import pallas as pl
from jax.experimental.pallas import tpu as pltpu
from jax.experimental.pallas import tpu_sc as plsc

N = 10000
D = 128
E = 320000

NC = 2
NS = 16
NW = NC * NS
CH = 128
CPT = 80
EP = NW * CPT * CH
NPAD = 512
NACC = N + NPAD
RPT = 624
LASTX = N - NS * RPT
ZR = 16
DEGW = 16


def _sc_mesh():
    return plsc.VectorSubcoreMesh(core_axis_name="c", subcore_axis_name="s")


def _zero_vmem(buf, rows, width):
    def row(i, carry):
        def col(j, c2):
            buf[i, pl.ds(j * 16, 16)] = jnp.zeros((16,), jnp.float32)
            return c2
        return lax.fori_loop(0, width // 16, col, carry)
    lax.fori_loop(0, rows, row, 0)


def _zero_acc(zb, acc, rbase, s):
    def zcp(i, carry):
        pltpu.sync_copy(zb, acc.at[pl.ds(rbase + i * ZR, ZR)])
        return carry
    lax.fori_loop(0, RPT // ZR, zcp, 0)

    @pl.when(s == NS - 1)
    def _():
        pltpu.sync_copy(zb, acc.at[pl.ds(rbase + RPT, LASTX)])


def _drain_acc(acc, out_hbm, c, rbase, s):
    pltpu.sync_copy(acc.at[pl.ds(rbase, RPT)],
                    out_hbm.at[c, pl.ds(rbase, RPT)])

    @pl.when(s == NS - 1)
    def _():
        pltpu.sync_copy(acc.at[pl.ds(rbase + RPT, LASTX)],
                        out_hbm.at[c, pl.ds(rbase + RPT, LASTX)])


def _sc_degree(dst2d):
    @functools.partial(
        pl.kernel,
        out_type=jax.ShapeDtypeStruct((NC, N, DEGW), jnp.float32),
        mesh=_sc_mesh(),
        scratch_types=[
            pltpu.VMEM((CPT, CH), jnp.int32),
            pltpu.VMEM((CH, DEGW), jnp.float32),
            pltpu.VMEM((ZR, DEGW), jnp.float32),
            pltpu.VMEM_SHARED((NACC, DEGW), jnp.float32),
            pltpu.SemaphoreType.DMA,
        ],
    )
    def deg_kernel(dst_hbm, out_hbm, didx, ones, zb, acc, sem):
        c = lax.axis_index("c")
        s = lax.axis_index("s")
        wid = c * NS + s
        rbase = s * RPT

        _zero_vmem(zb, ZR, DEGW)

        def ones_row(i, carry):
            ones[i, pl.ds(0, 16)] = jnp.ones((16,), jnp.float32)
            return carry
        lax.fori_loop(0, CH, ones_row, 0)

        pltpu.sync_copy(dst_hbm.at[pl.ds(wid * CPT, CPT)], didx)
        _zero_acc(zb, acc, rbase, s)
        plsc.subcore_barrier()

        def body(p, carry):
            pltpu.sync_copy(ones, acc.at[didx.at[p]], add=True)
            return carry
        lax.fori_loop(0, CPT, body, 0)
        plsc.subcore_barrier()

        _drain_acc(acc, out_hbm, c, rbase, s)

    return deg_kernel(dst2d)


def _sc_scatter(mp, src2d, dst2d):
    @functools.partial(
        pl.kernel,
        out_type=jax.ShapeDtypeStruct((NC, N, D), jnp.float32),
        mesh=_sc_mesh(),
        scratch_types=[
            [pltpu.VMEM((CH,), jnp.int32) for _ in range(4)],
            [pltpu.VMEM((CH,), jnp.int32) for _ in range(4)],
            pltpu.VMEM((CH, D), jnp.float32),
            pltpu.VMEM((CH, D), jnp.float32),
            pltpu.VMEM((ZR, D), jnp.float32),
            pltpu.VMEM_SHARED((NACC, D), jnp.float32),
            pltpu.SemaphoreType.DMA,
            pltpu.SemaphoreType.DMA,
            [pltpu.SemaphoreType.DMA for _ in range(4)],
        ],
    )
    def scat_kernel(mp_hbm, src_hbm, dst_hbm, out_hbm,
                    si, di, r0, r1, zb, acc, gs0, gs1, isem):
        c = lax.axis_index("c")
        s = lax.axis_index("s")
        wid = c * NS + s
        rbase = s * RPT

        def load_idx(j, q):
            pltpu.async_copy(src_hbm.at[wid * CPT + j], si[q], isem[q])
            pltpu.async_copy(dst_hbm.at[wid * CPT + j], di[q], isem[q])

        def wait_idx(j, q):
            pltpu.make_async_copy(src_hbm.at[wid * CPT + j], si[q],
                                  isem[q]).wait()
            pltpu.make_async_copy(dst_hbm.at[wid * CPT + j], di[q],
                                  isem[q]).wait()

        def gather(rb, q, gs):
            pltpu.async_copy(mp_hbm.at[si[q]], rb, gs)

        for q in range(4):
            load_idx(q, q)
        _zero_vmem(zb, ZR, D)
        _zero_acc(zb, acc, rbase, s)
        plsc.subcore_barrier()

        wait_idx(0, 0)
        gather(r0, 0, gs0)
        wait_idx(1, 1)
        gather(r1, 1, gs1)

        def handle(j, q, rb, gs):
            pltpu.make_async_copy(mp_hbm.at[si[q]], rb, gs).wait()
            pltpu.sync_copy(rb, acc.at[di[q]], add=True)

            @pl.when(j + 4 < CPT)
            def _():
                load_idx(j + 4, q)

            @pl.when(j + 2 < CPT)
            def _():
                wait_idx(j + 2, (q + 2) % 4)
                gather(rb, (q + 2) % 4, gs)

        def body(p, carry):
            j = 4 * p
            handle(j, 0, r0, gs0)
            handle(j + 1, 1, r1, gs1)
            handle(j + 2, 2, r0, gs0)
            handle(j + 3, 3, r1, gs1)
            return carry
        lax.fori_loop(0, CPT // 4, body, 0)
        plsc.subcore_barrier()

        _drain_acc(acc, out_hbm, c, rbase, s)

    return scat_kernel(mp, src2d, dst2d)


def _dinv_from(degp_ref):
    deg = degp_ref[0, :, 0] + degp_ref[1, :, 0] + 1.0
    return lax.rsqrt(deg)


def _tc_first(x, gamma, beta, W1, degp):
    def body(x_ref, g_ref, b_ref, w_ref, degp_ref, out_ref):
        xv = x_ref[...]
        mean = jnp.mean(xv, axis=0, keepdims=True)
        var = jnp.mean((xv - mean) ** 2, axis=0, keepdims=True)
        h = (xv - mean) / jnp.sqrt(var + 1e-5) * g_ref[...] + b_ref[...]
        dinv = _dinv_from(degp_ref)
        xw = jnp.dot(h, w_ref[...], preferred_element_type=jnp.float32)
        out_ref[...] = xw * dinv[:, None]

    return pl.pallas_call(
        body,
        out_shape=jax.ShapeDtypeStruct((N, D), jnp.float32),
    )(x, gamma.reshape(1, D), beta.reshape(1, D), W1, degp)


def _tc_mid(acc, mp, degp, b, W2):
    def body(acc_ref, mp_ref, degp_ref, b_ref, w_ref, out_ref):
        dinv = _dinv_from(degp_ref)
        tot = acc_ref[0] + acc_ref[1] + mp_ref[...]
        h = jnp.maximum(tot * dinv[:, None] + b_ref[...], 0.0)
        xw = jnp.dot(h, w_ref[...], preferred_element_type=jnp.float32)
        out_ref[...] = xw * dinv[:, None]

    return pl.pallas_call(
        body,
        out_shape=jax.ShapeDtypeStruct((N, D), jnp.float32),
    )(acc, mp, degp, b.reshape(1, D), W2)


def _tc_final(acc, mp, degp, b2, Wc1, bc1, Wc2, bc2):
    def body(acc_ref, mp_ref, degp_ref, b2_ref, wc1_ref, bc1_ref,
             wc2_ref, bc2_ref, out_ref):
        dinv = _dinv_from(degp_ref)
        tot = acc_ref[0] + acc_ref[1] + mp_ref[...]
        h = jnp.maximum(tot * dinv[:, None] + b2_ref[...], 0.0)
        hc = jnp.dot(h, wc1_ref[...], preferred_element_type=jnp.float32)
        hc = jnp.maximum(hc + bc1_ref[...], 0.0)
        out = jnp.dot(hc, wc2_ref[...], preferred_element_type=jnp.float32)
        out_ref[...] = out + bc2_ref[...]

    return pl.pallas_call(
        body,
        out_shape=jax.ShapeDtypeStruct((N, 2), jnp.float32),
    )(acc, mp, degp, b2.reshape(1, D), Wc1, bc1.reshape(1, 16),
      Wc2, bc2.reshape(1, 2))


def kernel(x, edge_index, bn_gamma, bn_beta, W1, b1, W2, b2, Wc1, bc1, Wc2, bc2):
    src = edge_index[0].astype(jnp.int32)
    dst = edge_index[1].astype(jnp.int32)

    pad = EP - E
    psrc = jnp.arange(pad, dtype=jnp.int32) % N
    pdst = N + (jnp.arange(pad, dtype=jnp.int32) % NPAD)
    src2d = jnp.concatenate([src, psrc]).reshape(EP // CH, CH)
    dst2d = jnp.concatenate([dst, pdst]).reshape(EP // CH, CH)

    degp = _sc_degree(dst2d)
    mp1 = _tc_first(x, bn_gamma, bn_beta, W1, degp)
    acc1 = _sc_scatter(mp1, src2d, dst2d)
    mp2 = _tc_mid(acc1, mp1, degp, b1, W2)
    acc2 = _sc_scatter(mp2, src2d, dst2d)
    return _tc_final(acc2, mp2, degp, b2, Wc1, bc1, Wc2, bc2)


# speedup vs baseline: 3.0515x; 1.0111x over previous
import functools

import jax
import jax.numpy as jnp
from jax import lax
from jax.experimental import pallas as pl
from jax.experimental.pallas import tpu as pltpu
from jax.experimental.pallas import tpu_sc as plsc

N = 10000
D = 128
E = 320000

NC = 2
NS = 16
NW = NC * NS
CH = 128
CPT = 80
EP = NW * CPT * CH
NPAD = 512
NACC = N + NPAD
RPT = 624
LASTX = N - NS * RPT
ZR = 16
DEGW = 16


def _sc_mesh():
    return plsc.VectorSubcoreMesh(core_axis_name="c", subcore_axis_name="s")


def _zero_vmem(buf, rows, width):
    def row(i, carry):
        def col(j, c2):
            buf[i, pl.ds(j * 16, 16)] = jnp.zeros((16,), jnp.float32)
            return c2
        return lax.fori_loop(0, width // 16, col, carry)
    lax.fori_loop(0, rows, row, 0)


def _zero_acc(zb, acc, rbase, s):
    def zcp(i, carry):
        pltpu.sync_copy(zb, acc.at[pl.ds(rbase + i * ZR, ZR)])
        return carry
    lax.fori_loop(0, RPT // ZR, zcp, 0)

    @pl.when(s == NS - 1)
    def _():
        pltpu.sync_copy(zb, acc.at[pl.ds(rbase + RPT, LASTX)])


def _drain_acc(acc, out_hbm, c, rbase, s):
    pltpu.sync_copy(acc.at[pl.ds(rbase, RPT)],
                    out_hbm.at[c, pl.ds(rbase, RPT)])

    @pl.when(s == NS - 1)
    def _():
        pltpu.sync_copy(acc.at[pl.ds(rbase + RPT, LASTX)],
                        out_hbm.at[c, pl.ds(rbase + RPT, LASTX)])


def _sc_degree(dst2d):
    @functools.partial(
        pl.kernel,
        out_type=jax.ShapeDtypeStruct((NC, N, DEGW), jnp.float32),
        mesh=_sc_mesh(),
        scratch_types=[
            pltpu.VMEM((CPT, CH), jnp.int32),
            pltpu.VMEM((CH, DEGW), jnp.float32),
            pltpu.VMEM((ZR, DEGW), jnp.float32),
            pltpu.VMEM_SHARED((NACC, DEGW), jnp.float32),
            pltpu.SemaphoreType.DMA,
        ],
    )
    def deg_kernel(dst_hbm, out_hbm, didx, ones, zb, acc, sem):
        c = lax.axis_index("c")
        s = lax.axis_index("s")
        wid = c * NS + s
        rbase = s * RPT

        _zero_vmem(zb, ZR, DEGW)

        def ones_row(i, carry):
            ones[i, pl.ds(0, 16)] = jnp.ones((16,), jnp.float32)
            return carry
        lax.fori_loop(0, CH, ones_row, 0)

        pltpu.sync_copy(dst_hbm.at[pl.ds(wid * CPT, CPT)], didx)
        _zero_acc(zb, acc, rbase, s)
        plsc.subcore_barrier()

        def body(p, carry):
            pltpu.sync_copy(ones, acc.at[didx.at[p]], add=True)
            return carry
        lax.fori_loop(0, CPT, body, 0)
        plsc.subcore_barrier()

        _drain_acc(acc, out_hbm, c, rbase, s)

    return deg_kernel(dst2d)


def _sc_scatter(mp, src2d, dst2d):
    @functools.partial(
        pl.kernel,
        out_type=jax.ShapeDtypeStruct((NC, N, D), jnp.float32),
        mesh=_sc_mesh(),
        scratch_types=[
            [pltpu.VMEM((CH,), jnp.int32) for _ in range(8)],
            [pltpu.VMEM((CH,), jnp.int32) for _ in range(8)],
            pltpu.VMEM((CH, D), jnp.float32),
            pltpu.VMEM((CH, D), jnp.float32),
            pltpu.VMEM((ZR, D), jnp.float32),
            pltpu.VMEM_SHARED((NACC, D), jnp.float32),
            [pltpu.SemaphoreType.DMA for _ in range(2)],
            [pltpu.SemaphoreType.DMA for _ in range(8)],
        ],
    )
    def scat_kernel(mp_hbm, src_hbm, dst_hbm, out_hbm,
                    si, di, r0, r1, zb, acc, gs, isem):
        c = lax.axis_index("c")
        s = lax.axis_index("s")
        wid = c * NS + s
        rbase = s * RPT
        r = [r0, r1]

        def load_idx(j, q):
            pltpu.async_copy(src_hbm.at[wid * CPT + j], si[q], isem[q])
            pltpu.async_copy(dst_hbm.at[wid * CPT + j], di[q], isem[q])

        def wait_idx(j, q):
            pltpu.make_async_copy(src_hbm.at[wid * CPT + j], si[q],
                                  isem[q]).wait()
            pltpu.make_async_copy(dst_hbm.at[wid * CPT + j], di[q],
                                  isem[q]).wait()

        def gather(b, q):
            pltpu.async_copy(mp_hbm.at[si[q]], r[b], gs[b])

        for q in range(4):
            load_idx(q, q)
        _zero_vmem(zb, ZR, D)
        wait_idx(0, 0)
        gather(0, 0)
        wait_idx(1, 1)
        gather(1, 1)
        _zero_acc(zb, acc, rbase, s)
        plsc.subcore_barrier()

        def handle(j, b, q):
            pltpu.make_async_copy(mp_hbm.at[si[q]], r[b], gs[b]).wait()
            pltpu.sync_copy(r[b], acc.at[di[q]], add=True)

            @pl.when(j + 4 < CPT)
            def _():
                load_idx(j + 4, (q + 4) % 8)

            @pl.when(j + 2 < CPT)
            def _():
                wait_idx(j + 2, (q + 2) % 8)
                gather(b, (q + 2) % 8)

        def body(p, carry):
            j = 8 * p
            for k in range(8):
                handle(j + k, k % 2, k)
            return carry
        lax.fori_loop(0, CPT // 8, body, 0)
        plsc.subcore_barrier()

        _drain_acc(acc, out_hbm, c, rbase, s)

    return scat_kernel(mp, src2d, dst2d)


def _dinv_from(degp_ref):
    deg = degp_ref[0, :, 0] + degp_ref[1, :, 0] + 1.0
    return lax.rsqrt(deg)


def _tc_first(x, gamma, beta, W1):
    def body(x_ref, g_ref, b_ref, w_ref, out_ref):
        xv = x_ref[...]
        mean = jnp.mean(xv, axis=0, keepdims=True)
        var = jnp.mean((xv - mean) ** 2, axis=0, keepdims=True)
        h = (xv - mean) / jnp.sqrt(var + 1e-5) * g_ref[...] + b_ref[...]
        out_ref[...] = jnp.dot(h, w_ref[...],
                               preferred_element_type=jnp.float32)

    return pl.pallas_call(
        body,
        out_shape=jax.ShapeDtypeStruct((N, D), jnp.float32),
    )(x, gamma.reshape(1, D), beta.reshape(1, D), W1)


def _tc_scale(xw, degp):
    def body(xw_ref, degp_ref, out_ref):
        dinv = _dinv_from(degp_ref)
        out_ref[...] = xw_ref[...] * dinv[:, None]

    return pl.pallas_call(
        body,
        out_shape=jax.ShapeDtypeStruct((N, D), jnp.float32),
    )(xw, degp)


def _tc_mid(acc, mp, degp, b, W2):
    def body(acc_ref, mp_ref, degp_ref, b_ref, w_ref, out_ref):
        dinv = _dinv_from(degp_ref)
        tot = acc_ref[0] + acc_ref[1] + mp_ref[...]
        h = jnp.maximum(tot * dinv[:, None] + b_ref[...], 0.0)
        xw = jnp.dot(h, w_ref[...], preferred_element_type=jnp.float32)
        out_ref[...] = xw * dinv[:, None]

    return pl.pallas_call(
        body,
        out_shape=jax.ShapeDtypeStruct((N, D), jnp.float32),
    )(acc, mp, degp, b.reshape(1, D), W2)


def _tc_final(acc, mp, degp, b2, Wc1, bc1, Wc2, bc2):
    def body(acc_ref, mp_ref, degp_ref, b2_ref, wc1_ref, bc1_ref,
             wc2_ref, bc2_ref, out_ref):
        dinv = _dinv_from(degp_ref)
        tot = acc_ref[0] + acc_ref[1] + mp_ref[...]
        h = jnp.maximum(tot * dinv[:, None] + b2_ref[...], 0.0)
        hc = jnp.dot(h, wc1_ref[...], preferred_element_type=jnp.float32)
        hc = jnp.maximum(hc + bc1_ref[...], 0.0)
        out = jnp.dot(hc, wc2_ref[...], preferred_element_type=jnp.float32)
        out_ref[...] = out + bc2_ref[...]

    return pl.pallas_call(
        body,
        out_shape=jax.ShapeDtypeStruct((N, 2), jnp.float32),
    )(acc, mp, degp, b2.reshape(1, D), Wc1, bc1.reshape(1, 16),
      Wc2, bc2.reshape(1, 2))


def kernel(x, edge_index, bn_gamma, bn_beta, W1, b1, W2, b2, Wc1, bc1, Wc2, bc2):
    src = edge_index[0].astype(jnp.int32)
    dst = edge_index[1].astype(jnp.int32)

    pad = EP - E
    psrc = jnp.arange(pad, dtype=jnp.int32) % N
    pdst = N + (jnp.arange(pad, dtype=jnp.int32) % NPAD)
    src2d = jnp.concatenate([src, psrc]).reshape(EP // CH, CH)
    dst2d = jnp.concatenate([dst, pdst]).reshape(EP // CH, CH)

    degp = _sc_degree(dst2d)
    xw1 = _tc_first(x, bn_gamma, bn_beta, W1)
    mp1 = _tc_scale(xw1, degp)
    acc1 = _sc_scatter(mp1, src2d, dst2d)
    mp2 = _tc_mid(acc1, mp1, degp, b1, W2)
    acc2 = _sc_scatter(mp2, src2d, dst2d)
    return _tc_final(acc2, mp2, degp, b2, Wc1, bc1, Wc2, bc2)


# speedup vs baseline: 3.0518x; 1.0001x over previous
import functools

import jax
import jax.numpy as jnp
from jax import lax
from jax.experimental import pallas as pl
from jax.experimental.pallas import tpu as pltpu
from jax.experimental.pallas import tpu_sc as plsc

N = 10000
D = 128
E = 320000

NC = 2
NS = 16
NW = NC * NS
CH = 128
CPT = 80
EP = NW * CPT * CH
NPAD = 512
NACC = N + NPAD
RPT = 624
LASTX = N - NS * RPT
ZR = 16
DEGW = 16


def _sc_mesh():
    return plsc.VectorSubcoreMesh(core_axis_name="c", subcore_axis_name="s")


def _zero_vmem(buf, rows, width):
    def row(i, carry):
        def col(j, c2):
            buf[i, pl.ds(j * 16, 16)] = jnp.zeros((16,), jnp.float32)
            return c2
        return lax.fori_loop(0, width // 16, col, carry)
    lax.fori_loop(0, rows, row, 0)


def _zero_acc(zb, acc, rbase, s):
    def zcp(i, carry):
        pltpu.sync_copy(zb, acc.at[pl.ds(rbase + i * ZR, ZR)])
        return carry
    lax.fori_loop(0, RPT // ZR, zcp, 0)

    @pl.when(s == NS - 1)
    def _():
        pltpu.sync_copy(zb, acc.at[pl.ds(rbase + RPT, LASTX)])


def _drain_acc(acc, out_hbm, c, rbase, s):
    pltpu.sync_copy(acc.at[pl.ds(rbase, RPT)],
                    out_hbm.at[c, pl.ds(rbase, RPT)])

    @pl.when(s == NS - 1)
    def _():
        pltpu.sync_copy(acc.at[pl.ds(rbase + RPT, LASTX)],
                        out_hbm.at[c, pl.ds(rbase + RPT, LASTX)])


def _sc_degree(dst2d):
    @functools.partial(
        pl.kernel,
        out_type=jax.ShapeDtypeStruct((NC, N, DEGW), jnp.float32),
        mesh=_sc_mesh(),
        scratch_types=[
            pltpu.VMEM((CPT, CH), jnp.int32),
            pltpu.VMEM((CH, DEGW), jnp.float32),
            pltpu.VMEM((ZR, DEGW), jnp.float32),
            pltpu.VMEM_SHARED((NACC, DEGW), jnp.float32),
            pltpu.SemaphoreType.DMA,
        ],
    )
    def deg_kernel(dst_hbm, out_hbm, didx, ones, zb, acc, sem):
        c = lax.axis_index("c")
        s = lax.axis_index("s")
        wid = c * NS + s
        rbase = s * RPT

        _zero_vmem(zb, ZR, DEGW)

        def ones_row(i, carry):
            ones[i, pl.ds(0, 16)] = jnp.ones((16,), jnp.float32)
            return carry
        lax.fori_loop(0, CH, ones_row, 0)

        pltpu.sync_copy(dst_hbm.at[pl.ds(wid * CPT, CPT)], didx)
        _zero_acc(zb, acc, rbase, s)
        plsc.subcore_barrier()

        def body(p, carry):
            pltpu.sync_copy(ones, acc.at[didx.at[p]], add=True)
            return carry
        lax.fori_loop(0, CPT, body, 0)
        plsc.subcore_barrier()

        _drain_acc(acc, out_hbm, c, rbase, s)

    return deg_kernel(dst2d)


def _sc_scatter(mp, src2d, dst2d):
    @functools.partial(
        pl.kernel,
        out_type=jax.ShapeDtypeStruct((NC, N, D), jnp.float32),
        mesh=_sc_mesh(),
        scratch_types=[
            [pltpu.VMEM((CH,), jnp.int32) for _ in range(8)],
            [pltpu.VMEM((CH,), jnp.int32) for _ in range(8)],
            pltpu.VMEM((CH, D), jnp.float32),
            pltpu.VMEM((CH, D), jnp.float32),
            pltpu.VMEM((ZR, D), jnp.float32),
            pltpu.VMEM_SHARED((NACC, D), jnp.float32),
            [pltpu.SemaphoreType.DMA for _ in range(2)],
            [pltpu.SemaphoreType.DMA for _ in range(8)],
        ],
    )
    def scat_kernel(mp_hbm, src_hbm, dst_hbm, out_hbm,
                    si, di, r0, r1, zb, acc, gs, isem):
        c = lax.axis_index("c")
        s = lax.axis_index("s")
        wid = c * NS + s
        rbase = s * RPT
        r = [r0, r1]

        def load_idx(j, q):
            pltpu.async_copy(src_hbm.at[wid * CPT + j], si[q], isem[q])
            pltpu.async_copy(dst_hbm.at[wid * CPT + j], di[q], isem[q])

        def wait_idx(j, q):
            pltpu.make_async_copy(src_hbm.at[wid * CPT + j], si[q],
                                  isem[q]).wait()
            pltpu.make_async_copy(dst_hbm.at[wid * CPT + j], di[q],
                                  isem[q]).wait()

        def gather(b, q):
            pltpu.async_copy(mp_hbm.at[si[q]], r[b], gs[b])

        for q in range(4):
            load_idx(q, q)
        _zero_vmem(zb, ZR, D)
        wait_idx(0, 0)
        gather(0, 0)
        wait_idx(1, 1)
        gather(1, 1)
        _zero_acc(zb, acc, rbase, s)
        plsc.subcore_barrier()

        def handle(j, b, q):
            pltpu.make_async_copy(mp_hbm.at[si[q]], r[b], gs[b]).wait()
            pltpu.sync_copy(r[b], acc.at[di[q]], add=True)

            @pl.when(j + 4 < CPT)
            def _():
                load_idx(j + 4, (q + 4) % 8)

            @pl.when(j + 2 < CPT)
            def _():
                wait_idx(j + 2, (q + 2) % 8)
                gather(b, (q + 2) % 8)

        def body(p, carry):
            j = 8 * p
            for k in range(8):
                handle(j + k, k % 2, k)
            return carry
        lax.fori_loop(0, CPT // 8, body, 0)
        plsc.subcore_barrier()

        _drain_acc(acc, out_hbm, c, rbase, s)

    return scat_kernel(mp, src2d, dst2d)


def _dinv_from(degp_ref):
    deg = degp_ref[0, :, 0] + degp_ref[1, :, 0] + 1.0
    return lax.rsqrt(deg)


def _tc_first(x, gamma, beta, W1, degp):
    def body(x_ref, g_ref, b_ref, w_ref, degp_ref, out_ref):
        xv = x_ref[...]
        mean = jnp.mean(xv, axis=0, keepdims=True)
        var = jnp.mean((xv - mean) ** 2, axis=0, keepdims=True)
        h = (xv - mean) / jnp.sqrt(var + 1e-5) * g_ref[...] + b_ref[...]
        dinv = _dinv_from(degp_ref)
        xw = jnp.dot(h, w_ref[...], preferred_element_type=jnp.float32)
        out_ref[...] = xw * dinv[:, None]

    return pl.pallas_call(
        body,
        out_shape=jax.ShapeDtypeStruct((N, D), jnp.float32),
    )(x, gamma.reshape(1, D), beta.reshape(1, D), W1, degp)


def _tc_mid(acc, mp, degp, b, W2):
    def body(acc_ref, mp_ref, degp_ref, b_ref, w_ref, out_ref):
        dinv = _dinv_from(degp_ref)
        tot = acc_ref[0] + acc_ref[1] + mp_ref[...]
        h = jnp.maximum(tot * dinv[:, None] + b_ref[...], 0.0)
        xw = jnp.dot(h, w_ref[...], preferred_element_type=jnp.float32)
        out_ref[...] = xw * dinv[:, None]

    return pl.pallas_call(
        body,
        out_shape=jax.ShapeDtypeStruct((N, D), jnp.float32),
    )(acc, mp, degp, b.reshape(1, D), W2)


def _tc_final(acc, mp, degp, b2, Wc1, bc1, Wc2, bc2):
    def body(acc_ref, mp_ref, degp_ref, b2_ref, wc1_ref, bc1_ref,
             wc2_ref, bc2_ref, out_ref):
        dinv = _dinv_from(degp_ref)
        tot = acc_ref[0] + acc_ref[1] + mp_ref[...]
        h = jnp.maximum(tot * dinv[:, None] + b2_ref[...], 0.0)
        hc = jnp.dot(h, wc1_ref[...], preferred_element_type=jnp.float32)
        hc = jnp.maximum(hc + bc1_ref[...], 0.0)
        out = jnp.dot(hc, wc2_ref[...], preferred_element_type=jnp.float32)
        out_ref[...] = out + bc2_ref[...]

    return pl.pallas_call(
        body,
        out_shape=jax.ShapeDtypeStruct((N, 2), jnp.float32),
    )(acc, mp, degp, b2.reshape(1, D), Wc1, bc1.reshape(1, 16),
      Wc2, bc2.reshape(1, 2))


def kernel(x, edge_index, bn_gamma, bn_beta, W1, b1, W2, b2, Wc1, bc1, Wc2, bc2):
    src = edge_index[0].astype(jnp.int32)
    dst = edge_index[1].astype(jnp.int32)

    pad = EP - E
    psrc = jnp.arange(pad, dtype=jnp.int32) % N
    pdst = N + (jnp.arange(pad, dtype=jnp.int32) % NPAD)
    src2d = jnp.concatenate([src, psrc]).reshape(EP // CH, CH)
    dst2d = jnp.concatenate([dst, pdst]).reshape(EP // CH, CH)

    degp = _sc_degree(dst2d)
    mp1 = _tc_first(x, bn_gamma, bn_beta, W1, degp)
    acc1 = _sc_scatter(mp1, src2d, dst2d)
    mp2 = _tc_mid(acc1, mp1, degp, b1, W2)
    acc2 = _sc_scatter(mp2, src2d, dst2d)
    return _tc_final(acc2, mp2, degp, b2, Wc1, bc1, Wc2, bc2)
